# initial kernel scaffold (unmeasured)
import jax
import jax.numpy as jnp
from jax import lax
from jax.experimental import pallas as pl
from jax.experimental.pallas import tpu as pltpu

N_DEV = 8


def kernel(x, w_mat):
    m_per, k = x.shape
    _, n_per = w_mat.shape

    def body(x_ref, w_ref, out_ref, comm_ref, send_sems, recv_sems):
        my = lax.axis_index("i")
        left = (my - 1) % N_DEV
        right = (my + 1) % N_DEV

        barrier_sem = pltpu.get_barrier_semaphore()
        for nbr in (left, right):
            pl.semaphore_signal(
                barrier_sem, inc=1,
                device_id=(nbr,), device_id_type=pl.DeviceIdType.MESH,
            )
        pl.semaphore_wait(barrier_sem, 2)

        def silu_mm(xc):
            y = jnp.dot(xc, w_ref[:, :], preferred_element_type=jnp.float32)
            return y * jax.nn.sigmoid(y)

        out_ref[pl.ds(my * m_per, m_per), :] = silu_mm(x_ref[:, :])

        for h in range(N_DEV - 1):
            src = x_ref if h == 0 else comm_ref.at[h - 1]
            rdma = pltpu.make_async_remote_copy(
                src_ref=src,
                dst_ref=comm_ref.at[h],
                send_sem=send_sems.at[h],
                recv_sem=recv_sems.at[h],
                device_id=(right,),
                device_id_type=pl.DeviceIdType.MESH,
            )
            rdma.start()
            rdma.wait()
            origin = (my - h - 1) % N_DEV
            out_ref[pl.ds(origin * m_per, m_per), :] = silu_mm(
                comm_ref[h, :, :]
            )

    return pl.pallas_call(
        body,
        out_shape=jax.ShapeDtypeStruct((N_DEV * m_per, n_per), jnp.float32),
        in_specs=[
            pl.BlockSpec(memory_space=pltpu.VMEM),
            pl.BlockSpec(memory_space=pltpu.VMEM),
        ],
        out_specs=pl.BlockSpec(memory_space=pltpu.VMEM),
        scratch_shapes=[
            pltpu.VMEM((N_DEV - 1, m_per, k), jnp.float32),
            pltpu.SemaphoreType.DMA((N_DEV - 1,)),
            pltpu.SemaphoreType.DMA((N_DEV - 1,)),
        ],
        compiler_params=pltpu.CompilerParams(collective_id=0),
    )(x, w_mat)


# baseline (device time: 676104 ns/iter reference)
import jax
import jax.numpy as jnp
from jax import lax
from jax.experimental import pallas as pl
from jax.experimental.pallas import tpu as pltpu

N_DEV = 8
N_SLOT = 4


def kernel(x, w_mat):
    m_per, k = x.shape
    _, n_per = w_mat.shape

    def body(x_ref, w_ref, out_ref, comm_ref, send_sems, recv_sems,
             credit_sem):
        my = lax.axis_index("i")
        left = (my - 1) % N_DEV
        right = (my + 1) % N_DEV

        barrier_sem = pltpu.get_barrier_semaphore()
        for nbr in (left, right):
            pl.semaphore_signal(
                barrier_sem, inc=1,
                device_id=(nbr,), device_id_type=pl.DeviceIdType.MESH,
            )
        pl.semaphore_wait(barrier_sem, 2)

        def silu_mm(xc):
            y = jnp.dot(xc, w_ref[:, :], preferred_element_type=jnp.float32)
            return y * jax.nn.sigmoid(y)

        out_ref[pl.ds(my * m_per, m_per), :] = silu_mm(x_ref[:, :])

        n_hops = N_DEV - 1
        for h in range(n_hops):
            slot = h % N_SLOT
            if h >= N_SLOT:
                pl.semaphore_wait(credit_sem, 1)
            src = x_ref if h == 0 else comm_ref.at[(h - 1) % N_SLOT]
            rdma = pltpu.make_async_remote_copy(
                src_ref=src,
                dst_ref=comm_ref.at[slot],
                send_sem=send_sems.at[slot],
                recv_sem=recv_sems.at[slot],
                device_id=(right,),
                device_id_type=pl.DeviceIdType.MESH,
            )
            rdma.start()
            rdma.wait()
            if 1 <= h and h - 1 < n_hops - N_SLOT:
                pl.semaphore_signal(
                    credit_sem, inc=1,
                    device_id=(left,), device_id_type=pl.DeviceIdType.MESH,
                )
            origin = (my - h - 1) % N_DEV
            out_ref[pl.ds(origin * m_per, m_per), :] = silu_mm(
                comm_ref[slot, :, :]
            )

    return pl.pallas_call(
        body,
        out_shape=jax.ShapeDtypeStruct((N_DEV * m_per, n_per), jnp.float32),
        in_specs=[
            pl.BlockSpec(memory_space=pltpu.VMEM),
            pl.BlockSpec(memory_space=pltpu.VMEM),
        ],
        out_specs=pl.BlockSpec(memory_space=pltpu.VMEM),
        scratch_shapes=[
            pltpu.VMEM((N_SLOT, m_per, k), jnp.float32),
            pltpu.SemaphoreType.DMA((N_SLOT,)),
            pltpu.SemaphoreType.DMA((N_SLOT,)),
            pltpu.SemaphoreType.REGULAR,
        ],
        compiler_params=pltpu.CompilerParams(
            collective_id=0,
            vmem_limit_bytes=100 * 1024 * 1024,
        ),
    )(x, w_mat)


# device time: 346220 ns/iter; 1.9528x vs baseline; 1.9528x over previous
import jax
import jax.numpy as jnp
from jax import lax
from jax.experimental import pallas as pl
from jax.experimental.pallas import tpu as pltpu

N_DEV = 8
N_SLOT = 4


def kernel(x, w_mat):
    m_per, k = x.shape
    _, n_per = w_mat.shape
    m_half = m_per // 2
    n_hops = N_DEV - 1

    def body(x_ref, w_ref, out_ref,
             comm_r, comm_l, send_r, recv_r, send_l, recv_l,
             credit_r, credit_l):
        my = lax.axis_index("i")
        left = (my - 1) % N_DEV
        right = (my + 1) % N_DEV

        barrier_sem = pltpu.get_barrier_semaphore()
        for nbr in (left, right):
            pl.semaphore_signal(
                barrier_sem, inc=1,
                device_id=(nbr,), device_id_type=pl.DeviceIdType.MESH,
            )
        pl.semaphore_wait(barrier_sem, 2)

        def silu_mm(xc):
            y = jnp.dot(xc, w_ref[:, :], preferred_element_type=jnp.float32)
            return y * jax.nn.sigmoid(y)

        def make(dirn, h):
            slot = h % N_SLOT
            if dirn == "r":
                src = (x_ref.at[pl.ds(0, m_half)] if h == 0
                       else comm_r.at[(h - 1) % N_SLOT])
                return pltpu.make_async_remote_copy(
                    src_ref=src, dst_ref=comm_r.at[slot],
                    send_sem=send_r.at[slot], recv_sem=recv_r.at[slot],
                    device_id=(right,), device_id_type=pl.DeviceIdType.MESH,
                )
            src = (x_ref.at[pl.ds(m_half, m_half)] if h == 0
                   else comm_l.at[(h - 1) % N_SLOT])
            return pltpu.make_async_remote_copy(
                src_ref=src, dst_ref=comm_l.at[slot],
                send_sem=send_l.at[slot], recv_sem=recv_l.at[slot],
                device_id=(left,), device_id_type=pl.DeviceIdType.MESH,
            )

        rdmas = {}
        for d in ("r", "l"):
            rdmas[(d, 0)] = make(d, 0)
            rdmas[(d, 0)].start()

        out_ref[pl.ds(my * m_per, m_per), :] = silu_mm(x_ref[:, :])

        for h in range(n_hops):
            for d in ("r", "l"):
                rdmas[(d, h)].wait_recv()
            for d in ("r", "l"):
                rdmas[(d, h)].wait_send()
            if 1 <= h <= n_hops - N_SLOT:
                pl.semaphore_signal(
                    credit_r, inc=1,
                    device_id=(left,), device_id_type=pl.DeviceIdType.MESH,
                )
                pl.semaphore_signal(
                    credit_l, inc=1,
                    device_id=(right,), device_id_type=pl.DeviceIdType.MESH,
                )
            if h < n_hops - 1:
                if h + 1 >= N_SLOT:
                    pl.semaphore_wait(credit_r, 1)
                    pl.semaphore_wait(credit_l, 1)
                for d in ("r", "l"):
                    rdmas[(d, h + 1)] = make(d, h + 1)
                    rdmas[(d, h + 1)].start()
            origin_r = (my - h - 1) % N_DEV
            origin_l = (my + h + 1) % N_DEV
            out_ref[pl.ds(origin_r * m_per, m_half), :] = silu_mm(
                comm_r[h % N_SLOT, :, :]
            )
            out_ref[pl.ds(origin_l * m_per + m_half, m_half), :] = silu_mm(
                comm_l[h % N_SLOT, :, :]
            )

    return pl.pallas_call(
        body,
        out_shape=jax.ShapeDtypeStruct((N_DEV * m_per, n_per), jnp.float32),
        in_specs=[
            pl.BlockSpec(memory_space=pltpu.VMEM),
            pl.BlockSpec(memory_space=pltpu.VMEM),
        ],
        out_specs=pl.BlockSpec(memory_space=pltpu.VMEM),
        scratch_shapes=[
            pltpu.VMEM((N_SLOT, m_half, k), jnp.float32),
            pltpu.VMEM((N_SLOT, m_half, k), jnp.float32),
            pltpu.SemaphoreType.DMA((N_SLOT,)),
            pltpu.SemaphoreType.DMA((N_SLOT,)),
            pltpu.SemaphoreType.DMA((N_SLOT,)),
            pltpu.SemaphoreType.DMA((N_SLOT,)),
            pltpu.SemaphoreType.REGULAR,
            pltpu.SemaphoreType.REGULAR,
        ],
        compiler_params=pltpu.CompilerParams(
            collective_id=0,
            vmem_limit_bytes=100 * 1024 * 1024,
        ),
    )(x, w_mat)


# device time: 292126 ns/iter; 2.3144x vs baseline; 1.1852x over previous
import jax
import jax.numpy as jnp
from jax import lax
from jax.experimental import pallas as pl
from jax.experimental.pallas import tpu as pltpu

N_DEV = 8
N_PLANE = 4


def kernel(x, w_mat):
    m_per, k = x.shape
    _, n_per = w_mat.shape
    m_half = m_per // 2

    def body(x_ref, w_ref, out_ref, pair_buf,
             own_r, own_l, par_r, par_l,
             s_own_r, r_own_r, s_own_l, r_own_l,
             s_par_r, r_par_r, s_par_l, r_par_l,
             s_z, r_z,
             c_own_r, c_own_l, c_par_r, c_par_l):
        my = lax.axis_index("i")
        p = my % N_PLANE
        base = my - p
        right = base + (p + 1) % N_PLANE
        left = base + (p - 1) % N_PLANE
        partner = (my + 4) % N_DEV

        barrier_sem = pltpu.get_barrier_semaphore()
        for nbr in (left, right, partner):
            pl.semaphore_signal(
                barrier_sem, inc=1,
                device_id=(nbr,), device_id_type=pl.DeviceIdType.MESH,
            )
        pl.semaphore_wait(barrier_sem, 3)

        def silu_mm(xc):
            y = jnp.dot(xc, w_ref[:, :], preferred_element_type=jnp.float32)
            return y * jax.nn.sigmoid(y)

        def copy(src, dst, send_sem, recv_sem, dev):
            return pltpu.make_async_remote_copy(
                src_ref=src, dst_ref=dst, send_sem=send_sem,
                recv_sem=recv_sem, device_id=(dev,),
                device_id_type=pl.DeviceIdType.MESH,
            )

        def credit(sem, dev):
            pl.semaphore_signal(
                sem, inc=1, device_id=(dev,),
                device_id_type=pl.DeviceIdType.MESH,
            )

        z = copy(x_ref, pair_buf, s_z, r_z, partner)
        z.start()
        r1 = copy(x_ref.at[pl.ds(0, m_half)], own_r.at[0],
                  s_own_r.at[0], r_own_r.at[0], right)
        l1 = copy(x_ref.at[pl.ds(m_half, m_half)], own_l.at[0],
                  s_own_l.at[0], r_own_l.at[0], left)
        r1.start()
        l1.start()
        out_ref[pl.ds(my * m_per, m_per), :] = silu_mm(x_ref[:, :])

        r1.wait_recv()
        l1.wait_recv()
        r2 = copy(own_r.at[0], own_r.at[1], s_own_r.at[1], r_own_r.at[1],
                  right)
        l2 = copy(own_l.at[0], own_l.at[1], s_own_l.at[1], r_own_l.at[1],
                  left)
        r2.start()
        l2.start()
        z.wait_recv()
        pr1 = copy(pair_buf.at[pl.ds(0, m_half)], par_r.at[0],
                   s_par_r.at[0], r_par_r.at[0], right)
        pl1 = copy(pair_buf.at[pl.ds(m_half, m_half)], par_l.at[0],
                   s_par_l.at[0], r_par_l.at[0], left)
        pr1.start()
        pl1.start()
        z.wait_send()
        out_ref[pl.ds(partner * m_per, m_per), :] = silu_mm(pair_buf[:, :])
        o = base + (p - 1) % N_PLANE
        out_ref[pl.ds(o * m_per, m_half), :] = silu_mm(own_r[0, :, :])
        o = base + (p + 1) % N_PLANE
        out_ref[pl.ds(o * m_per + m_half, m_half), :] = silu_mm(
            own_l[0, :, :])

        r2.wait_recv()
        l2.wait_recv()
        r1.wait_send()
        l1.wait_send()
        r2.wait_send()
        l2.wait_send()
        credit(c_own_r, left)
        credit(c_own_l, right)
        pl.semaphore_wait(c_own_r, 1)
        pl.semaphore_wait(c_own_l, 1)
        r3 = copy(own_r.at[1], own_r.at[0], s_own_r.at[0], r_own_r.at[0],
                  right)
        l3 = copy(own_l.at[1], own_l.at[0], s_own_l.at[0], r_own_l.at[0],
                  left)
        r3.start()
        l3.start()
        pr1.wait_recv()
        pl1.wait_recv()
        pr2 = copy(par_r.at[0], par_r.at[1], s_par_r.at[1], r_par_r.at[1],
                   right)
        pl2 = copy(par_l.at[0], par_l.at[1], s_par_l.at[1], r_par_l.at[1],
                   left)
        pr2.start()
        pl2.start()
        o = base + (p - 2) % N_PLANE
        out_ref[pl.ds(o * m_per, m_half), :] = silu_mm(own_r[1, :, :])
        o = base + (p + 2) % N_PLANE
        out_ref[pl.ds(o * m_per + m_half, m_half), :] = silu_mm(
            own_l[1, :, :])
        o = (base + (p - 1) % N_PLANE + 4) % N_DEV
        out_ref[pl.ds(o * m_per, m_half), :] = silu_mm(par_r[0, :, :])
        o = (base + (p + 1) % N_PLANE + 4) % N_DEV
        out_ref[pl.ds(o * m_per + m_half, m_half), :] = silu_mm(
            par_l[0, :, :])

        r3.wait_recv()
        l3.wait_recv()
        pr2.wait_recv()
        pl2.wait_recv()
        pr1.wait_send()
        pl1.wait_send()
        pr2.wait_send()
        pl2.wait_send()
        credit(c_par_r, left)
        credit(c_par_l, right)
        pl.semaphore_wait(c_par_r, 1)
        pl.semaphore_wait(c_par_l, 1)
        pr3 = copy(par_r.at[1], par_r.at[0], s_par_r.at[0], r_par_r.at[0],
                   right)
        pl3 = copy(par_l.at[1], par_l.at[0], s_par_l.at[0], r_par_l.at[0],
                   left)
        pr3.start()
        pl3.start()
        r3.wait_send()
        l3.wait_send()
        o = base + (p - 3) % N_PLANE
        out_ref[pl.ds(o * m_per, m_half), :] = silu_mm(own_r[0, :, :])
        o = base + (p + 3) % N_PLANE
        out_ref[pl.ds(o * m_per + m_half, m_half), :] = silu_mm(
            own_l[0, :, :])
        o = (base + (p - 2) % N_PLANE + 4) % N_DEV
        out_ref[pl.ds(o * m_per, m_half), :] = silu_mm(par_r[1, :, :])
        o = (base + (p + 2) % N_PLANE + 4) % N_DEV
        out_ref[pl.ds(o * m_per + m_half, m_half), :] = silu_mm(
            par_l[1, :, :])

        pr3.wait_recv()
        pl3.wait_recv()
        pr3.wait_send()
        pl3.wait_send()
        o = (base + (p - 3) % N_PLANE + 4) % N_DEV
        out_ref[pl.ds(o * m_per, m_half), :] = silu_mm(par_r[0, :, :])
        o = (base + (p + 3) % N_PLANE + 4) % N_DEV
        out_ref[pl.ds(o * m_per + m_half, m_half), :] = silu_mm(
            par_l[0, :, :])

    dma2 = pltpu.SemaphoreType.DMA((2,))
    return pl.pallas_call(
        body,
        out_shape=jax.ShapeDtypeStruct((N_DEV * m_per, n_per), jnp.float32),
        in_specs=[
            pl.BlockSpec(memory_space=pltpu.VMEM),
            pl.BlockSpec(memory_space=pltpu.VMEM),
        ],
        out_specs=pl.BlockSpec(memory_space=pltpu.VMEM),
        scratch_shapes=[
            pltpu.VMEM((m_per, k), jnp.float32),
            pltpu.VMEM((2, m_half, k), jnp.float32),
            pltpu.VMEM((2, m_half, k), jnp.float32),
            pltpu.VMEM((2, m_half, k), jnp.float32),
            pltpu.VMEM((2, m_half, k), jnp.float32),
            dma2, dma2,
            dma2, dma2,
            dma2, dma2,
            dma2, dma2,
            pltpu.SemaphoreType.DMA, pltpu.SemaphoreType.DMA,
            pltpu.SemaphoreType.REGULAR,
            pltpu.SemaphoreType.REGULAR,
            pltpu.SemaphoreType.REGULAR,
            pltpu.SemaphoreType.REGULAR,
        ],
        compiler_params=pltpu.CompilerParams(
            collective_id=0,
            vmem_limit_bytes=100 * 1024 * 1024,
        ),
    )(x, w_mat)


# device time: 250988 ns/iter; 2.6938x vs baseline; 1.1639x over previous
import jax
import jax.numpy as jnp
from jax import lax
from jax.experimental import pallas as pl
from jax.experimental.pallas import tpu as pltpu

N_DEV = 8
N_PLANE = 4


def kernel(x, w_mat):
    m_per, k = x.shape
    _, n_per = w_mat.shape
    m_half = m_per // 2

    def body(x_ref, w_ref, out_ref, pair_buf, diag_top, diag_bot,
             own_r, own_l, par_r, par_l,
             s_own_r, r_own_r, s_own_l, r_own_l,
             s_par_r, r_par_r, s_par_l, r_par_l,
             s_z, r_z, s_zt, r_zt, s_zb, r_zb,
             c_own_r, c_own_l, c_par_r, c_par_l):
        my = lax.axis_index("i")
        p = my % N_PLANE
        base = my - p
        right = base + (p + 1) % N_PLANE
        left = base + (p - 1) % N_PLANE
        partner = (my + 4) % N_DEV

        barrier_sem = pltpu.get_barrier_semaphore()
        for nbr in (left, right, partner):
            pl.semaphore_signal(
                barrier_sem, inc=1,
                device_id=(nbr,), device_id_type=pl.DeviceIdType.MESH,
            )
        pl.semaphore_wait(barrier_sem, 3)

        def silu_mm(xc):
            y = jnp.dot(xc, w_ref[:, :], preferred_element_type=jnp.float32)
            return y * jax.nn.sigmoid(y)

        def copy(src, dst, send_sem, recv_sem, dev):
            return pltpu.make_async_remote_copy(
                src_ref=src, dst_ref=dst, send_sem=send_sem,
                recv_sem=recv_sem, device_id=(dev,),
                device_id_type=pl.DeviceIdType.MESH,
            )

        def credit(sem, dev):
            pl.semaphore_signal(
                sem, inc=1, device_id=(dev,),
                device_id_type=pl.DeviceIdType.MESH,
            )

        def store_top(origin, buf):
            out_ref[pl.ds(origin * m_per, m_half), :] = silu_mm(buf)

        def store_bot(origin, buf):
            out_ref[pl.ds(origin * m_per + m_half, m_half), :] = silu_mm(buf)

        z1 = copy(x_ref, pair_buf, s_z, r_z, partner)
        z1.start()
        r1 = copy(x_ref.at[pl.ds(0, m_half)], own_r.at[0],
                  s_own_r.at[0], r_own_r.at[0], right)
        l1 = copy(x_ref.at[pl.ds(m_half, m_half)], own_l.at[0],
                  s_own_l.at[0], r_own_l.at[0], left)
        r1.start()
        l1.start()
        out_ref[pl.ds(my * m_per, m_per), :] = silu_mm(x_ref[:, :])

        r1.wait_recv()
        l1.wait_recv()
        r2 = copy(own_r.at[0], own_r.at[1], s_own_r.at[1], r_own_r.at[1],
                  right)
        l2 = copy(own_l.at[0], own_l.at[1], s_own_l.at[1], r_own_l.at[1],
                  left)
        r2.start()
        l2.start()
        z1.wait_recv()
        pr1 = copy(pair_buf.at[pl.ds(0, m_half)], par_r.at[0],
                   s_par_r.at[0], r_par_r.at[0], right)
        pl1 = copy(pair_buf.at[pl.ds(m_half, m_half)], par_l.at[0],
                   s_par_l.at[0], r_par_l.at[0], left)
        pr1.start()
        pl1.start()
        z1.wait_send()
        out_ref[pl.ds(partner * m_per, m_per), :] = silu_mm(pair_buf[:, :])
        store_top(base + (p - 1) % N_PLANE, own_r[0, :, :])
        store_bot(base + (p + 1) % N_PLANE, own_l[0, :, :])

        r2.wait_recv()
        l2.wait_recv()
        r1.wait_send()
        l1.wait_send()
        r2.wait_send()
        l2.wait_send()
        credit(c_own_r, left)
        credit(c_own_l, right)
        pl.semaphore_wait(c_own_r, 1)
        pl.semaphore_wait(c_own_l, 1)
        r3 = copy(own_r.at[1], own_r.at[0], s_own_r.at[0], r_own_r.at[0],
                  right)
        l3 = copy(own_l.at[1], own_l.at[0], s_own_l.at[0], r_own_l.at[0],
                  left)
        r3.start()
        l3.start()
        zt = copy(own_r.at[1], diag_top, s_zt, r_zt, partner)
        zb = copy(own_l.at[1], diag_bot, s_zb, r_zb, partner)
        zt.start()
        zb.start()
        store_top(base + (p - 2) % N_PLANE, own_r[1, :, :])
        store_bot(base + (p + 2) % N_PLANE, own_l[1, :, :])
        pr1.wait_recv()
        pl1.wait_recv()
        pr1.wait_send()
        pl1.wait_send()
        store_top((base + (p - 1) % N_PLANE + 4) % N_DEV, par_r[0, :, :])
        store_bot((base + (p + 1) % N_PLANE + 4) % N_DEV, par_l[0, :, :])
        credit(c_par_r, left)
        credit(c_par_l, right)

        zt.wait_recv()
        pl.semaphore_wait(c_par_r, 1)
        pr3 = copy(diag_top, par_r.at[0], s_par_r.at[1], r_par_r.at[1],
                   right)
        pr3.start()
        zb.wait_recv()
        pl.semaphore_wait(c_par_l, 1)
        pl3 = copy(diag_bot, par_l.at[0], s_par_l.at[1], r_par_l.at[1],
                   left)
        pl3.start()
        diag_p = (base + (p + 2) % N_PLANE + 4) % N_DEV
        store_top(diag_p, diag_top[:, :])
        store_bot(diag_p, diag_bot[:, :])
        r3.wait_recv()
        l3.wait_recv()
        r3.wait_send()
        l3.wait_send()
        store_top(base + (p - 3) % N_PLANE, own_r[0, :, :])
        store_bot(base + (p + 3) % N_PLANE, own_l[0, :, :])
        zt.wait_send()
        zb.wait_send()

        pr3.wait_recv()
        pl3.wait_recv()
        pr3.wait_send()
        pl3.wait_send()
        store_top((base + (p - 3) % N_PLANE + 4) % N_DEV, par_r[0, :, :])
        store_bot((base + (p + 3) % N_PLANE + 4) % N_DEV, par_l[0, :, :])

    dma1 = pltpu.SemaphoreType.DMA
    dma2 = pltpu.SemaphoreType.DMA((2,))
    return pl.pallas_call(
        body,
        out_shape=jax.ShapeDtypeStruct((N_DEV * m_per, n_per), jnp.float32),
        in_specs=[
            pl.BlockSpec(memory_space=pltpu.VMEM),
            pl.BlockSpec(memory_space=pltpu.VMEM),
        ],
        out_specs=pl.BlockSpec(memory_space=pltpu.VMEM),
        scratch_shapes=[
            pltpu.VMEM((m_per, k), jnp.float32),
            pltpu.VMEM((m_half, k), jnp.float32),
            pltpu.VMEM((m_half, k), jnp.float32),
            pltpu.VMEM((2, m_half, k), jnp.float32),
            pltpu.VMEM((2, m_half, k), jnp.float32),
            pltpu.VMEM((1, m_half, k), jnp.float32),
            pltpu.VMEM((1, m_half, k), jnp.float32),
            dma2, dma2,
            dma2, dma2,
            dma2, dma2,
            dma2, dma2,
            dma1, dma1,
            dma1, dma1,
            dma1, dma1,
            pltpu.SemaphoreType.REGULAR,
            pltpu.SemaphoreType.REGULAR,
            pltpu.SemaphoreType.REGULAR,
            pltpu.SemaphoreType.REGULAR,
        ],
        compiler_params=pltpu.CompilerParams(
            collective_id=0,
            vmem_limit_bytes=100 * 1024 * 1024,
        ),
    )(x, w_mat)


# device time: 239721 ns/iter; 2.8204x vs baseline; 1.0470x over previous
import jax
import jax.numpy as jnp
from jax import lax
from jax.experimental import pallas as pl
from jax.experimental.pallas import tpu as pltpu

N_DEV = 8
N_PLANE = 4
SPL_Z = 88
SPL_P = 168


def kernel(x, w_mat):
    m_per, k = x.shape
    _, n_per = w_mat.shape
    m_half = m_per // 2

    def body(x_ref, w_ref, out_ref, pair_buf, diag_top, diag_bot,
             relay_top, relay_bot,
             own_r, own_l, par_r, par_l,
             s_own_r, r_own_r, s_own_l, r_own_l,
             s_par_r, r_par_r, s_par_l, r_par_l,
             s_z, r_z, s_zt, r_zt, s_zb, r_zb,
             s_z3t, r_z3t, s_z3b, r_z3b,
             c_own_r, c_own_l, c_par_r, c_par_l):
        my = lax.axis_index("i")
        p = my % N_PLANE
        base = my - p
        right = base + (p + 1) % N_PLANE
        left = base + (p - 1) % N_PLANE
        partner = (my + 4) % N_DEV

        barrier_sem = pltpu.get_barrier_semaphore()
        for nbr in (left, right, partner):
            pl.semaphore_signal(
                barrier_sem, inc=1,
                device_id=(nbr,), device_id_type=pl.DeviceIdType.MESH,
            )
        pl.semaphore_wait(barrier_sem, 3)

        def silu_mm(xc):
            y = jnp.dot(xc, w_ref[:, :], preferred_element_type=jnp.float32)
            return y * jax.nn.sigmoid(y)

        def copy(src, dst, send_sem, recv_sem, dev):
            return pltpu.make_async_remote_copy(
                src_ref=src, dst_ref=dst, send_sem=send_sem,
                recv_sem=recv_sem, device_id=(dev,),
                device_id_type=pl.DeviceIdType.MESH,
            )

        def credit(sem, dev):
            pl.semaphore_signal(
                sem, inc=1, device_id=(dev,),
                device_id_type=pl.DeviceIdType.MESH,
            )

        def store(row0, buf):
            out_ref[pl.ds(row0, buf.shape[0]), :] = silu_mm(buf)

        z1 = copy(x_ref, pair_buf, s_z, r_z, partner)
        z1.start()
        r1 = copy(x_ref.at[pl.ds(0, m_half)], own_r.at[0],
                  s_own_r.at[0], r_own_r.at[0], right)
        l1 = copy(x_ref.at[pl.ds(m_half, m_half)], own_l.at[0],
                  s_own_l.at[0], r_own_l.at[0], left)
        r1.start()
        l1.start()
        store(my * m_per, x_ref[:, :])

        r1.wait_recv()
        l1.wait_recv()
        r2 = copy(own_r.at[0], own_r.at[1], s_own_r.at[1], r_own_r.at[1],
                  right)
        l2 = copy(own_l.at[0], own_l.at[1], s_own_l.at[1], r_own_l.at[1],
                  left)
        r2.start()
        l2.start()
        z1.wait_recv()
        pr1 = copy(pair_buf.at[pl.ds(0, m_half)], par_r,
                   s_par_r.at[0], r_par_r.at[0], right)
        pl1 = copy(pair_buf.at[pl.ds(m_half, m_half)], par_l,
                   s_par_l.at[0], r_par_l.at[0], left)
        pr1.start()
        pl1.start()
        z1.wait_send()
        store(partner * m_per, pair_buf[:, :])
        store((base + (p - 1) % N_PLANE) * m_per, own_r[0, :, :])
        store((base + (p + 1) % N_PLANE) * m_per + m_half, own_l[0, :, :])

        r2.wait_recv()
        l2.wait_recv()
        r1.wait_send()
        l1.wait_send()
        r2.wait_send()
        l2.wait_send()
        credit(c_own_r, left)
        credit(c_own_l, right)
        pl.semaphore_wait(c_own_r, 1)
        pl.semaphore_wait(c_own_l, 1)
        r3 = copy(own_r.at[1], own_r.at[0], s_own_r.at[0], r_own_r.at[0],
                  right)
        l3 = copy(own_l.at[1], own_l.at[0], s_own_l.at[0], r_own_l.at[0],
                  left)
        r3.start()
        l3.start()
        zt = copy(own_r.at[1], diag_top, s_zt, r_zt, partner)
        zb = copy(own_l.at[1], diag_bot, s_zb, r_zb, partner)
        zt.start()
        zb.start()
        store((base + (p - 2) % N_PLANE) * m_per, own_r[1, :, :])
        store((base + (p + 2) % N_PLANE) * m_per + m_half, own_l[1, :, :])
        pr1.wait_recv()
        pl1.wait_recv()
        pr1.wait_send()
        pl1.wait_send()
        store(((base + (p - 1) % N_PLANE + 4) % N_DEV) * m_per,
              par_r[:, :])
        store(((base + (p + 1) % N_PLANE + 4) % N_DEV) * m_per + m_half,
              par_l[:, :])
        credit(c_par_r, left)
        credit(c_par_l, right)

        zt.wait_recv()
        pl.semaphore_wait(c_par_r, 1)
        pr3 = copy(diag_top.at[pl.ds(SPL_Z, SPL_P)],
                   par_r.at[pl.ds(SPL_Z, SPL_P)],
                   s_par_r.at[1], r_par_r.at[1], right)
        pr3.start()
        zb.wait_recv()
        pl.semaphore_wait(c_par_l, 1)
        pl3 = copy(diag_bot.at[pl.ds(SPL_Z, SPL_P)],
                   par_l.at[pl.ds(SPL_Z, SPL_P)],
                   s_par_l.at[1], r_par_l.at[1], left)
        pl3.start()
        diag_p = (base + (p + 2) % N_PLANE + 4) % N_DEV
        store(diag_p * m_per, diag_top[:, :])
        store(diag_p * m_per + m_half, diag_bot[:, :])
        r3.wait_recv()
        l3.wait_recv()
        z3t = copy(own_r.at[0].at[pl.ds(0, SPL_Z)], relay_top,
                   s_z3t, r_z3t, partner)
        z3b = copy(own_l.at[0].at[pl.ds(0, SPL_Z)], relay_bot,
                   s_z3b, r_z3b, partner)
        z3t.start()
        z3b.start()
        store((base + (p - 3) % N_PLANE) * m_per, own_r[0, :, :])
        store((base + (p + 3) % N_PLANE) * m_per + m_half, own_l[0, :, :])
        r3.wait_send()
        l3.wait_send()
        zt.wait_send()
        zb.wait_send()

        rp = (base + (p - 3) % N_PLANE + 4) % N_DEV
        lp = (base + (p + 3) % N_PLANE + 4) % N_DEV
        z3t.wait_recv()
        store(rp * m_per, relay_top[:, :])
        z3b.wait_recv()
        store(lp * m_per + m_half, relay_bot[:, :])
        pr3.wait_recv()
        store(rp * m_per + SPL_Z, par_r[SPL_Z:m_half, :])
        pl3.wait_recv()
        store(lp * m_per + m_half + SPL_Z, par_l[SPL_Z:m_half, :])
        pr3.wait_send()
        pl3.wait_send()
        z3t.wait_send()
        z3b.wait_send()

    dma1 = pltpu.SemaphoreType.DMA
    dma2 = pltpu.SemaphoreType.DMA((2,))
    return pl.pallas_call(
        body,
        out_shape=jax.ShapeDtypeStruct((N_DEV * m_per, n_per), jnp.float32),
        in_specs=[
            pl.BlockSpec(memory_space=pltpu.VMEM),
            pl.BlockSpec(memory_space=pltpu.VMEM),
        ],
        out_specs=pl.BlockSpec(memory_space=pltpu.VMEM),
        scratch_shapes=[
            pltpu.VMEM((m_per, k), jnp.float32),
            pltpu.VMEM((m_half, k), jnp.float32),
            pltpu.VMEM((m_half, k), jnp.float32),
            pltpu.VMEM((SPL_Z, k), jnp.float32),
            pltpu.VMEM((SPL_Z, k), jnp.float32),
            pltpu.VMEM((2, m_half, k), jnp.float32),
            pltpu.VMEM((2, m_half, k), jnp.float32),
            pltpu.VMEM((m_half, k), jnp.float32),
            pltpu.VMEM((m_half, k), jnp.float32),
            dma2, dma2,
            dma2, dma2,
            dma2, dma2,
            dma2, dma2,
            dma1, dma1,
            dma1, dma1,
            dma1, dma1,
            dma1, dma1,
            dma1, dma1,
            pltpu.SemaphoreType.REGULAR,
            pltpu.SemaphoreType.REGULAR,
            pltpu.SemaphoreType.REGULAR,
            pltpu.SemaphoreType.REGULAR,
        ],
        compiler_params=pltpu.CompilerParams(
            collective_id=0,
            vmem_limit_bytes=100 * 1024 * 1024,
        ),
    )(x, w_mat)


# device time: 236606 ns/iter; 2.8575x vs baseline; 1.0132x over previous
import jax
import jax.numpy as jnp
from jax import lax
from jax.experimental import pallas as pl
from jax.experimental.pallas import tpu as pltpu

N_DEV = 8
N_PLANE = 4
QTR = 128
SPL_Z = 72
SPL_P = 184


def kernel(x, w_mat):
    m_per, k = x.shape
    _, n_per = w_mat.shape
    m_half = m_per // 2

    def body(x_ref, w_ref, out_ref, pair_buf, diag_top, diag_bot,
             relay_top, relay_bot,
             own_r, own_l, par_r, par_l,
             s_own_r, r_own_r, s_own_l, r_own_l,
             s_par_r, r_par_r, s_par_l, r_par_l,
             s_z, r_z, s_zta, r_zta, s_ztb, r_ztb,
             s_zba, r_zba, s_zbb, r_zbb,
             s_z3t, r_z3t, s_z3b, r_z3b,
             c_own_r, c_own_l, c_par_r, c_par_l):
        my = lax.axis_index("i")
        p = my % N_PLANE
        base = my - p
        right = base + (p + 1) % N_PLANE
        left = base + (p - 1) % N_PLANE
        partner = (my + 4) % N_DEV

        barrier_sem = pltpu.get_barrier_semaphore()
        for nbr in (left, right, partner):
            pl.semaphore_signal(
                barrier_sem, inc=1,
                device_id=(nbr,), device_id_type=pl.DeviceIdType.MESH,
            )
        pl.semaphore_wait(barrier_sem, 3)

        def silu_mm(xc):
            y = jnp.dot(xc, w_ref[:, :], preferred_element_type=jnp.float32)
            return y * jax.nn.sigmoid(y)

        def copy(src, dst, send_sem, recv_sem, dev):
            return pltpu.make_async_remote_copy(
                src_ref=src, dst_ref=dst, send_sem=send_sem,
                recv_sem=recv_sem, device_id=(dev,),
                device_id_type=pl.DeviceIdType.MESH,
            )

        def credit(sem, dev):
            pl.semaphore_signal(
                sem, inc=1, device_id=(dev,),
                device_id_type=pl.DeviceIdType.MESH,
            )

        def store(row0, buf):
            out_ref[pl.ds(row0, buf.shape[0]), :] = silu_mm(buf)

        z1 = copy(x_ref, pair_buf, s_z, r_z, partner)
        z1.start()
        r1 = copy(x_ref.at[pl.ds(0, m_half)], own_r.at[0],
                  s_own_r.at[0], r_own_r.at[0], right)
        l1 = copy(x_ref.at[pl.ds(m_half, m_half)], own_l.at[0],
                  s_own_l.at[0], r_own_l.at[0], left)
        r1.start()
        l1.start()
        store(my * m_per, x_ref[:, :])

        r1.wait_recv()
        l1.wait_recv()
        r2a = copy(own_r.at[0].at[pl.ds(0, QTR)],
                   own_r.at[1].at[pl.ds(0, QTR)],
                   s_own_r.at[1], r_own_r.at[1], right)
        r2b = copy(own_r.at[0].at[pl.ds(QTR, QTR)],
                   own_r.at[1].at[pl.ds(QTR, QTR)],
                   s_own_r.at[2], r_own_r.at[2], right)
        l2a = copy(own_l.at[0].at[pl.ds(0, QTR)],
                   own_l.at[1].at[pl.ds(0, QTR)],
                   s_own_l.at[1], r_own_l.at[1], left)
        l2b = copy(own_l.at[0].at[pl.ds(QTR, QTR)],
                   own_l.at[1].at[pl.ds(QTR, QTR)],
                   s_own_l.at[2], r_own_l.at[2], left)
        r2a.start()
        r2b.start()
        l2a.start()
        l2b.start()
        z1.wait_recv()
        pr1 = copy(pair_buf.at[pl.ds(0, m_half)], par_r,
                   s_par_r.at[0], r_par_r.at[0], right)
        pl1 = copy(pair_buf.at[pl.ds(m_half, m_half)], par_l,
                   s_par_l.at[0], r_par_l.at[0], left)
        pr1.start()
        pl1.start()
        z1.wait_send()
        store(partner * m_per, pair_buf[:, :])
        store((base + (p - 1) % N_PLANE) * m_per, own_r[0, :, :])
        store((base + (p + 1) % N_PLANE) * m_per + m_half, own_l[0, :, :])

        r2a.wait_recv()
        zta = copy(own_r.at[1].at[pl.ds(0, QTR)],
                   diag_top.at[pl.ds(0, QTR)], s_zta, r_zta, partner)
        zta.start()
        r2b.wait_recv()
        ztb = copy(own_r.at[1].at[pl.ds(QTR, QTR)],
                   diag_top.at[pl.ds(QTR, QTR)], s_ztb, r_ztb, partner)
        ztb.start()
        l2a.wait_recv()
        zba = copy(own_l.at[1].at[pl.ds(0, QTR)],
                   diag_bot.at[pl.ds(0, QTR)], s_zba, r_zba, partner)
        zba.start()
        l2b.wait_recv()
        zbb = copy(own_l.at[1].at[pl.ds(QTR, QTR)],
                   diag_bot.at[pl.ds(QTR, QTR)], s_zbb, r_zbb, partner)
        zbb.start()
        r1.wait_send()
        l1.wait_send()
        r2a.wait_send()
        r2b.wait_send()
        l2a.wait_send()
        l2b.wait_send()
        credit(c_own_r, left)
        credit(c_own_l, right)
        pl.semaphore_wait(c_own_r, 1)
        pl.semaphore_wait(c_own_l, 1)
        r3 = copy(own_r.at[1], own_r.at[0], s_own_r.at[0], r_own_r.at[0],
                  right)
        l3 = copy(own_l.at[1], own_l.at[0], s_own_l.at[0], r_own_l.at[0],
                  left)
        r3.start()
        l3.start()
        store((base + (p - 2) % N_PLANE) * m_per, own_r[1, :, :])
        store((base + (p + 2) % N_PLANE) * m_per + m_half, own_l[1, :, :])
        pr1.wait_recv()
        pl1.wait_recv()
        pr1.wait_send()
        pl1.wait_send()
        store(((base + (p - 1) % N_PLANE + 4) % N_DEV) * m_per,
              par_r[:, :])
        store(((base + (p + 1) % N_PLANE + 4) % N_DEV) * m_per + m_half,
              par_l[:, :])
        credit(c_par_r, left)
        credit(c_par_l, right)

        zta.wait_recv()
        ztb.wait_recv()
        pl.semaphore_wait(c_par_r, 1)
        pr3 = copy(diag_top.at[pl.ds(SPL_Z, SPL_P)],
                   par_r.at[pl.ds(SPL_Z, SPL_P)],
                   s_par_r.at[1], r_par_r.at[1], right)
        pr3.start()
        zba.wait_recv()
        zbb.wait_recv()
        pl.semaphore_wait(c_par_l, 1)
        pl3 = copy(diag_bot.at[pl.ds(SPL_Z, SPL_P)],
                   par_l.at[pl.ds(SPL_Z, SPL_P)],
                   s_par_l.at[1], r_par_l.at[1], left)
        pl3.start()
        diag_p = (base + (p + 2) % N_PLANE + 4) % N_DEV
        store(diag_p * m_per, diag_top[:, :])
        store(diag_p * m_per + m_half, diag_bot[:, :])
        r3.wait_recv()
        l3.wait_recv()
        zta.wait_send()
        ztb.wait_send()
        zba.wait_send()
        zbb.wait_send()
        z3t = copy(own_r.at[0].at[pl.ds(0, SPL_Z)], relay_top,
                   s_z3t, r_z3t, partner)
        z3b = copy(own_l.at[0].at[pl.ds(0, SPL_Z)], relay_bot,
                   s_z3b, r_z3b, partner)
        z3t.start()
        z3b.start()
        store((base + (p - 3) % N_PLANE) * m_per, own_r[0, :, :])
        store((base + (p + 3) % N_PLANE) * m_per + m_half, own_l[0, :, :])
        r3.wait_send()
        l3.wait_send()

        rp = (base + (p - 3) % N_PLANE + 4) % N_DEV
        lp = (base + (p + 3) % N_PLANE + 4) % N_DEV
        z3t.wait_recv()
        store(rp * m_per, relay_top[:, :])
        z3b.wait_recv()
        store(lp * m_per + m_half, relay_bot[:, :])
        pr3.wait_recv()
        store(rp * m_per + SPL_Z, par_r[SPL_Z:m_half, :])
        pl3.wait_recv()
        store(lp * m_per + m_half + SPL_Z, par_l[SPL_Z:m_half, :])
        pr3.wait_send()
        pl3.wait_send()
        z3t.wait_send()
        z3b.wait_send()

    dma1 = pltpu.SemaphoreType.DMA
    dma2 = pltpu.SemaphoreType.DMA((2,))
    dma3 = pltpu.SemaphoreType.DMA((3,))
    return pl.pallas_call(
        body,
        out_shape=jax.ShapeDtypeStruct((N_DEV * m_per, n_per), jnp.float32),
        in_specs=[
            pl.BlockSpec(memory_space=pltpu.VMEM),
            pl.BlockSpec(memory_space=pltpu.VMEM),
        ],
        out_specs=pl.BlockSpec(memory_space=pltpu.VMEM),
        scratch_shapes=[
            pltpu.VMEM((m_per, k), jnp.float32),
            pltpu.VMEM((m_half, k), jnp.float32),
            pltpu.VMEM((m_half, k), jnp.float32),
            pltpu.VMEM((SPL_Z, k), jnp.float32),
            pltpu.VMEM((SPL_Z, k), jnp.float32),
            pltpu.VMEM((2, m_half, k), jnp.float32),
            pltpu.VMEM((2, m_half, k), jnp.float32),
            pltpu.VMEM((m_half, k), jnp.float32),
            pltpu.VMEM((m_half, k), jnp.float32),
            dma3, dma3,
            dma3, dma3,
            dma2, dma2,
            dma2, dma2,
            dma1, dma1,
            dma1, dma1,
            dma1, dma1,
            dma1, dma1,
            dma1, dma1,
            dma1, dma1,
            dma1, dma1,
            pltpu.SemaphoreType.REGULAR,
            pltpu.SemaphoreType.REGULAR,
            pltpu.SemaphoreType.REGULAR,
            pltpu.SemaphoreType.REGULAR,
        ],
        compiler_params=pltpu.CompilerParams(
            collective_id=0,
            vmem_limit_bytes=100 * 1024 * 1024,
        ),
    )(x, w_mat)


# device time: 234717 ns/iter; 2.8805x vs baseline; 1.0080x over previous
import jax
import jax.numpy as jnp
from jax import lax
from jax.experimental import pallas as pl
from jax.experimental.pallas import tpu as pltpu

N_DEV = 8
N_PLANE = 4
QTR = 128
SPL_Z = 88
SPL_P = 168


def kernel(x, w_mat):
    m_per, k = x.shape
    _, n_per = w_mat.shape
    m_half = m_per // 2

    def body(x_ref, w_ref, out_ref, pair_buf, diag_top, diag_bot,
             relay_top, relay_bot,
             own_r, own_l, par_r, par_l,
             s_own_r, r_own_r, s_own_l, r_own_l,
             s_par_r, r_par_r, s_par_l, r_par_l,
             s_z, r_z, s_zta, r_zta, s_ztb, r_ztb,
             s_zba, r_zba, s_zbb, r_zbb,
             s_z3t, r_z3t, s_z3b, r_z3b,
             c_own_r, c_own_l, c_par_r, c_par_l):
        my = lax.axis_index("i")
        p = my % N_PLANE
        base = my - p
        right = base + (p + 1) % N_PLANE
        left = base + (p - 1) % N_PLANE
        partner = (my + 4) % N_DEV

        barrier_sem = pltpu.get_barrier_semaphore()
        for nbr in (left, right, partner):
            pl.semaphore_signal(
                barrier_sem, inc=1,
                device_id=(nbr,), device_id_type=pl.DeviceIdType.MESH,
            )
        pl.semaphore_wait(barrier_sem, 3)

        def silu_mm(xc):
            y = jnp.dot(xc, w_ref[:, :], preferred_element_type=jnp.float32)
            return y * jax.nn.sigmoid(y)

        def copy(src, dst, send_sem, recv_sem, dev):
            return pltpu.make_async_remote_copy(
                src_ref=src, dst_ref=dst, send_sem=send_sem,
                recv_sem=recv_sem, device_id=(dev,),
                device_id_type=pl.DeviceIdType.MESH,
            )

        def credit(sem, dev):
            pl.semaphore_signal(
                sem, inc=1, device_id=(dev,),
                device_id_type=pl.DeviceIdType.MESH,
            )

        def store(row0, buf):
            out_ref[pl.ds(row0, buf.shape[0]), :] = silu_mm(buf)

        z1 = copy(x_ref, pair_buf, s_z, r_z, partner)
        z1.start()
        r1 = copy(x_ref.at[pl.ds(0, m_half)], own_r.at[0],
                  s_own_r.at[0], r_own_r.at[0], right)
        l1 = copy(x_ref.at[pl.ds(m_half, m_half)], own_l.at[0],
                  s_own_l.at[0], r_own_l.at[0], left)
        r1.start()
        l1.start()
        store(my * m_per, x_ref[:, :])

        r1.wait_recv()
        l1.wait_recv()
        r2a = copy(own_r.at[0].at[pl.ds(0, QTR)],
                   own_r.at[1].at[pl.ds(0, QTR)],
                   s_own_r.at[1], r_own_r.at[1], right)
        r2b = copy(own_r.at[0].at[pl.ds(QTR, QTR)],
                   own_r.at[1].at[pl.ds(QTR, QTR)],
                   s_own_r.at[2], r_own_r.at[2], right)
        l2a = copy(own_l.at[0].at[pl.ds(0, QTR)],
                   own_l.at[1].at[pl.ds(0, QTR)],
                   s_own_l.at[1], r_own_l.at[1], left)
        l2b = copy(own_l.at[0].at[pl.ds(QTR, QTR)],
                   own_l.at[1].at[pl.ds(QTR, QTR)],
                   s_own_l.at[2], r_own_l.at[2], left)
        r2a.start()
        r2b.start()
        l2a.start()
        l2b.start()
        z1.wait_recv()
        pr1 = copy(pair_buf.at[pl.ds(0, m_half)], par_r,
                   s_par_r.at[0], r_par_r.at[0], right)
        pl1 = copy(pair_buf.at[pl.ds(m_half, m_half)], par_l,
                   s_par_l.at[0], r_par_l.at[0], left)
        pr1.start()
        pl1.start()
        z1.wait_send()

        r2a.wait_recv()
        zta = copy(own_r.at[1].at[pl.ds(0, QTR)],
                   diag_top.at[pl.ds(0, QTR)], s_zta, r_zta, partner)
        zta.start()
        r2b.wait_recv()
        ztb = copy(own_r.at[1].at[pl.ds(QTR, QTR)],
                   diag_top.at[pl.ds(QTR, QTR)], s_ztb, r_ztb, partner)
        ztb.start()
        l2a.wait_recv()
        zba = copy(own_l.at[1].at[pl.ds(0, QTR)],
                   diag_bot.at[pl.ds(0, QTR)], s_zba, r_zba, partner)
        zba.start()
        l2b.wait_recv()
        zbb = copy(own_l.at[1].at[pl.ds(QTR, QTR)],
                   diag_bot.at[pl.ds(QTR, QTR)], s_zbb, r_zbb, partner)
        zbb.start()
        store(partner * m_per, pair_buf[:, :])
        store((base + (p - 1) % N_PLANE) * m_per, own_r[0, :, :])
        store((base + (p + 1) % N_PLANE) * m_per + m_half, own_l[0, :, :])
        r1.wait_send()
        l1.wait_send()
        r2a.wait_send()
        r2b.wait_send()
        l2a.wait_send()
        l2b.wait_send()
        credit(c_own_r, left)
        credit(c_own_l, right)
        pl.semaphore_wait(c_own_r, 1)
        pl.semaphore_wait(c_own_l, 1)
        r3 = copy(own_r.at[1], own_r.at[0], s_own_r.at[0], r_own_r.at[0],
                  right)
        l3 = copy(own_l.at[1], own_l.at[0], s_own_l.at[0], r_own_l.at[0],
                  left)
        r3.start()
        l3.start()
        store((base + (p - 2) % N_PLANE) * m_per, own_r[1, :, :])
        store((base + (p + 2) % N_PLANE) * m_per + m_half, own_l[1, :, :])
        pr1.wait_recv()
        pl1.wait_recv()
        pr1.wait_send()
        pl1.wait_send()
        store(((base + (p - 1) % N_PLANE + 4) % N_DEV) * m_per,
              par_r[:, :])
        store(((base + (p + 1) % N_PLANE + 4) % N_DEV) * m_per + m_half,
              par_l[:, :])
        credit(c_par_r, left)
        credit(c_par_l, right)

        zta.wait_recv()
        ztb.wait_recv()
        pl.semaphore_wait(c_par_r, 1)
        pr3 = copy(diag_top.at[pl.ds(SPL_Z, SPL_P)],
                   par_r.at[pl.ds(SPL_Z, SPL_P)],
                   s_par_r.at[1], r_par_r.at[1], right)
        pr3.start()
        zba.wait_recv()
        zbb.wait_recv()
        pl.semaphore_wait(c_par_l, 1)
        pl3 = copy(diag_bot.at[pl.ds(SPL_Z, SPL_P)],
                   par_l.at[pl.ds(SPL_Z, SPL_P)],
                   s_par_l.at[1], r_par_l.at[1], left)
        pl3.start()
        diag_p = (base + (p + 2) % N_PLANE + 4) % N_DEV
        store(diag_p * m_per, diag_top[:, :])
        store(diag_p * m_per + m_half, diag_bot[:, :])
        r3.wait_recv()
        l3.wait_recv()
        zta.wait_send()
        ztb.wait_send()
        zba.wait_send()
        zbb.wait_send()
        z3t = copy(own_r.at[0].at[pl.ds(0, SPL_Z)], relay_top,
                   s_z3t, r_z3t, partner)
        z3b = copy(own_l.at[0].at[pl.ds(0, SPL_Z)], relay_bot,
                   s_z3b, r_z3b, partner)
        z3t.start()
        z3b.start()
        store((base + (p - 3) % N_PLANE) * m_per, own_r[0, :, :])
        store((base + (p + 3) % N_PLANE) * m_per + m_half, own_l[0, :, :])
        r3.wait_send()
        l3.wait_send()

        rp = (base + (p - 3) % N_PLANE + 4) % N_DEV
        lp = (base + (p + 3) % N_PLANE + 4) % N_DEV
        z3t.wait_recv()
        store(rp * m_per, relay_top[:, :])
        z3b.wait_recv()
        store(lp * m_per + m_half, relay_bot[:, :])
        pr3.wait_recv()
        store(rp * m_per + SPL_Z, par_r[SPL_Z:m_half, :])
        pl3.wait_recv()
        store(lp * m_per + m_half + SPL_Z, par_l[SPL_Z:m_half, :])
        pr3.wait_send()
        pl3.wait_send()
        z3t.wait_send()
        z3b.wait_send()

    dma1 = pltpu.SemaphoreType.DMA
    dma2 = pltpu.SemaphoreType.DMA((2,))
    dma3 = pltpu.SemaphoreType.DMA((3,))
    return pl.pallas_call(
        body,
        out_shape=jax.ShapeDtypeStruct((N_DEV * m_per, n_per), jnp.float32),
        in_specs=[
            pl.BlockSpec(memory_space=pltpu.VMEM),
            pl.BlockSpec(memory_space=pltpu.VMEM),
        ],
        out_specs=pl.BlockSpec(memory_space=pltpu.VMEM),
        scratch_shapes=[
            pltpu.VMEM((m_per, k), jnp.float32),
            pltpu.VMEM((m_half, k), jnp.float32),
            pltpu.VMEM((m_half, k), jnp.float32),
            pltpu.VMEM((SPL_Z, k), jnp.float32),
            pltpu.VMEM((SPL_Z, k), jnp.float32),
            pltpu.VMEM((2, m_half, k), jnp.float32),
            pltpu.VMEM((2, m_half, k), jnp.float32),
            pltpu.VMEM((m_half, k), jnp.float32),
            pltpu.VMEM((m_half, k), jnp.float32),
            dma3, dma3,
            dma3, dma3,
            dma2, dma2,
            dma2, dma2,
            dma1, dma1,
            dma1, dma1,
            dma1, dma1,
            dma1, dma1,
            dma1, dma1,
            dma1, dma1,
            dma1, dma1,
            pltpu.SemaphoreType.REGULAR,
            pltpu.SemaphoreType.REGULAR,
            pltpu.SemaphoreType.REGULAR,
            pltpu.SemaphoreType.REGULAR,
        ],
        compiler_params=pltpu.CompilerParams(
            collective_id=0,
            vmem_limit_bytes=100 * 1024 * 1024,
        ),
    )(x, w_mat)
